# baseline (speedup 1.0000x reference)
import functools

import jax
import jax.numpy as jnp
from jax import lax
from jax.experimental import pallas as pl
from jax.experimental.pallas import tpu as pltpu
from jax.experimental.pallas import tpu_sc as plsc

_INPUT_DIM = 1000000
_EMBED_DIM = 32
_SCALE = float(_INPUT_DIM) ** 0.5

_NC = 2
_NS = 16
_NW = _NC * _NS
_CHUNK = 128


def _build_sc_gather(num_idx: int):
    assert num_idx % (_NW * _CHUNK) == 0
    per_w = num_idx // _NW
    n_chunks = per_w // _CHUNK
    mesh = plsc.VectorSubcoreMesh(core_axis_name="c", subcore_axis_name="s")

    @functools.partial(
        pl.kernel,
        mesh=mesh,
        out_type=jax.ShapeDtypeStruct((num_idx, _EMBED_DIM), jnp.float32),
        scratch_types=[
            pltpu.VMEM((n_chunks, _CHUNK), jnp.int32),
            pltpu.VMEM((_CHUNK, _EMBED_DIM), jnp.float32),
            pltpu.SemaphoreType.DMA,
        ],
        compiler_params=pltpu.CompilerParams(use_tc_tiling_on_sc=False),
    )
    def k(table_hbm, idx_hbm, out_hbm, idx_v, rows_v, sem):
        wid = lax.axis_index("s") * _NC + lax.axis_index("c")
        pltpu.sync_copy(idx_hbm.at[wid], idx_v)
        base = wid * per_w

        @pl.loop(0, n_chunks)
        def _(j):
            pltpu.async_copy(table_hbm.at[idx_v.at[j]], rows_v, sem).wait()

            @pl.loop(0, _CHUNK)
            def _(r):
                rows_v[r, pl.ds(0, 16)] = rows_v[r, pl.ds(0, 16)] * _SCALE
                rows_v[r, pl.ds(16, 16)] = rows_v[r, pl.ds(16, 16)] * _SCALE

            pltpu.sync_copy(rows_v, out_hbm.at[pl.ds(base + j * _CHUNK, _CHUNK)])

    return k


def kernel(x, table):
    b, s = x.shape
    num_idx = b * s
    idx = x.reshape(_NW, num_idx // (_NW * _CHUNK), _CHUNK).astype(jnp.int32)
    out = _build_sc_gather(num_idx)(table, idx)
    return out.reshape(b, s, _EMBED_DIM)


# speedup vs baseline: 1.1540x; 1.1540x over previous
import functools

import jax
import jax.numpy as jnp
from jax import lax
from jax.experimental import pallas as pl
from jax.experimental.pallas import tpu as pltpu
from jax.experimental.pallas import tpu_sc as plsc

_INPUT_DIM = 1000000
_EMBED_DIM = 32
_SCALE = float(_INPUT_DIM) ** 0.5

_NC = 2
_NS = 16
_NW = _NC * _NS
_CHUNK = 128
_NBUF = 4


def _build_sc_gather(num_idx: int):
    assert num_idx % (_NW * _CHUNK * _NBUF) == 0
    per_w = num_idx // _NW
    n_chunks = per_w // _CHUNK
    n_rounds = n_chunks // _NBUF
    mesh = plsc.VectorSubcoreMesh(core_axis_name="c", subcore_axis_name="s")

    @functools.partial(
        pl.kernel,
        mesh=mesh,
        out_type=jax.ShapeDtypeStruct((num_idx, _EMBED_DIM), jnp.float32),
        scratch_types=[
            pltpu.VMEM((n_chunks, _CHUNK), jnp.int32),
            [pltpu.VMEM((_CHUNK, _EMBED_DIM), jnp.float32)] * _NBUF,
            [pltpu.VMEM((_CHUNK, _EMBED_DIM), jnp.float32)] * _NBUF,
            [pltpu.SemaphoreType.DMA] * _NBUF,
            [pltpu.SemaphoreType.DMA] * _NBUF,
        ],
        compiler_params=pltpu.CompilerParams(use_tc_tiling_on_sc=False),
    )
    def k(table_hbm, idx_hbm, out_hbm, idx_v, grows, orows, gsem, osem):
        wid = lax.axis_index("s") * _NC + lax.axis_index("c")
        pltpu.sync_copy(idx_hbm.at[wid], idx_v)
        base = wid * per_w

        def start_gather(b, c):
            pltpu.async_copy(table_hbm.at[idx_v.at[c]], grows[b], gsem[b])

        def wait_gather(b):
            pltpu.make_async_copy(
                table_hbm.at[pl.ds(0, _CHUNK)], grows[b], gsem[b]
            ).wait()

        def start_out(b, c):
            pltpu.async_copy(
                orows[b], out_hbm.at[pl.ds(base + c * _CHUNK, _CHUNK)], osem[b]
            )

        def wait_out(b):
            pltpu.make_async_copy(
                orows[b], out_hbm.at[pl.ds(0, _CHUNK)], osem[b]
            ).wait()

        def scale(b):
            @pl.loop(0, _CHUNK, step=4)
            def _(r):
                for rr in range(4):
                    for h in range(0, _EMBED_DIM, 16):
                        orows[b][r + rr, pl.ds(h, 16)] = (
                            grows[b][r + rr, pl.ds(h, 16)] * _SCALE
                        )

        for b in range(_NBUF):
            start_gather(b, b)

        for b in range(_NBUF):
            wait_gather(b)
            scale(b)
            start_gather(b, _NBUF + b)
            start_out(b, b)

        @pl.loop(1, n_rounds - 1)
        def _(g):
            for b in range(_NBUF):
                c = g * _NBUF + b
                wait_gather(b)
                wait_out(b)
                scale(b)
                start_gather(b, c + _NBUF)
                start_out(b, c)

        for b in range(_NBUF):
            c = (n_rounds - 1) * _NBUF + b
            wait_gather(b)
            wait_out(b)
            scale(b)
            start_out(b, c)

        for b in range(_NBUF):
            wait_out(b)

    return k


def kernel(x, table):
    b, s = x.shape
    num_idx = b * s
    idx = x.reshape(_NW, num_idx // (_NW * _CHUNK), _CHUNK).astype(jnp.int32)
    out = _build_sc_gather(num_idx)(table, idx)
    return out.reshape(b, s, _EMBED_DIM)


# speedup vs baseline: 1.8633x; 1.6147x over previous
import functools

import jax
import jax.numpy as jnp
from jax import lax
from jax.experimental import pallas as pl
from jax.experimental.pallas import tpu as pltpu
from jax.experimental.pallas import tpu_sc as plsc

_INPUT_DIM = 1000000
_EMBED_DIM = 32
_SCALE = float(_INPUT_DIM) ** 0.5

_NC = 2
_NS = 16
_NW = _NC * _NS
_NBUF = 8


def _build_sc_gather(n_rows: int, row_len: int):
    assert n_rows % (_NW * _NBUF) == 0
    rows_per_w = n_rows // _NW
    n_rounds = rows_per_w // _NBUF
    mesh = plsc.VectorSubcoreMesh(core_axis_name="c", subcore_axis_name="s")

    @functools.partial(
        pl.kernel,
        mesh=mesh,
        out_type=jax.ShapeDtypeStruct((n_rows, row_len, _EMBED_DIM), jnp.float32),
        scratch_types=[
            pltpu.VMEM((rows_per_w, row_len), jnp.int32),
            [pltpu.VMEM((row_len, _EMBED_DIM), jnp.float32)] * _NBUF,
            [pltpu.VMEM((row_len, _EMBED_DIM), jnp.float32)] * _NBUF,
            [pltpu.SemaphoreType.DMA] * _NBUF,
            [pltpu.SemaphoreType.DMA] * _NBUF,
        ],
        compiler_params=pltpu.CompilerParams(use_tc_tiling_on_sc=False),
    )
    def k(table_hbm, idx_hbm, out_hbm, idx_v, grows, orows, gsem, osem):
        wid = lax.axis_index("s") * _NC + lax.axis_index("c")
        base = wid * rows_per_w
        pltpu.sync_copy(idx_hbm.at[pl.ds(base, rows_per_w)], idx_v)

        def start_gather(b, c):
            pltpu.async_copy(table_hbm.at[idx_v.at[c]], grows[b], gsem[b])

        def wait_gather(b):
            pltpu.make_async_copy(
                table_hbm.at[pl.ds(0, row_len)], grows[b], gsem[b]
            ).wait()

        def start_out(b, c):
            pltpu.async_copy(orows[b], out_hbm.at[base + c], osem[b])

        def wait_out(b):
            pltpu.make_async_copy(orows[b], out_hbm.at[0], osem[b]).wait()

        def scale(b):
            @pl.loop(0, row_len, step=2)
            def _(r):
                for rr in range(2):
                    for h in range(0, _EMBED_DIM, 16):
                        orows[b][r + rr, pl.ds(h, 16)] = (
                            grows[b][r + rr, pl.ds(h, 16)] * _SCALE
                        )

        for b in range(_NBUF):
            start_gather(b, b)

        for b in range(_NBUF):
            wait_gather(b)
            scale(b)
            start_gather(b, _NBUF + b)
            start_out(b, b)

        @pl.loop(1, n_rounds - 1)
        def _(g):
            for b in range(_NBUF):
                c = g * _NBUF + b
                wait_gather(b)
                wait_out(b)
                scale(b)
                start_gather(b, c + _NBUF)
                start_out(b, c)

        for b in range(_NBUF):
            c = (n_rounds - 1) * _NBUF + b
            wait_gather(b)
            wait_out(b)
            scale(b)
            start_out(b, c)

        for b in range(_NBUF):
            wait_out(b)

    return k


def kernel(x, table):
    b, s = x.shape
    out = _build_sc_gather(b, s)(table, x.astype(jnp.int32))
    return out
